# Initial kernel scaffold; baseline (speedup 1.0000x reference)
import jax
import jax.numpy as jnp
from jax.experimental import pallas as pl


def kernel(x, W):
    raise NotImplementedError("write your pallas kernel here")


# speedup vs baseline: 1.2921x; 1.2921x over previous
import functools

import jax
import jax.numpy as jnp
from jax.experimental import pallas as pl

DIM = 2048
N_EXPERTS = 8
BLOCK_N = 1024


def _gate_block(x_ref, wt_ref, w_ref, i_ref):
    bn = x_ref.shape[0]
    scores = jnp.dot(x_ref[...], wt_ref[...], preferred_element_type=jnp.float32)
    m = jnp.max(scores, axis=1, keepdims=True)
    e = jnp.exp(scores - m)
    p = e / jnp.sum(e, axis=1, keepdims=True)
    lane = jax.lax.broadcasted_iota(jnp.int32, (bn, N_EXPERTS), 1)
    w1 = jnp.max(p, axis=1, keepdims=True)
    i1 = jnp.min(jnp.where(p == w1, lane, N_EXPERTS), axis=1, keepdims=True)
    q = jnp.where(lane == i1, -1.0, p)
    w2 = jnp.max(q, axis=1, keepdims=True)
    i2 = jnp.min(jnp.where(q == w2, lane, N_EXPERTS), axis=1, keepdims=True)
    w_ref[...] = jnp.concatenate([w1, w2], axis=1)
    i_ref[...] = jnp.concatenate([i1, i2], axis=1)


@functools.partial(jax.jit, static_argnames=("interpret",))
def kernel(x, W, interpret=False):
    n_tokens = x.shape[0]
    wt = W.T
    grid = (n_tokens // BLOCK_N,)
    weights, indices = pl.pallas_call(
        _gate_block,
        grid=grid,
        in_specs=[
            pl.BlockSpec((BLOCK_N, DIM), lambda i: (i, 0)),
            pl.BlockSpec((DIM, N_EXPERTS), lambda i: (0, 0)),
        ],
        out_specs=[
            pl.BlockSpec((BLOCK_N, 2), lambda i: (i, 0)),
            pl.BlockSpec((BLOCK_N, 2), lambda i: (i, 0)),
        ],
        out_shape=[
            jax.ShapeDtypeStruct((n_tokens, 2), jnp.float32),
            jax.ShapeDtypeStruct((n_tokens, 2), jnp.int32),
        ],
        interpret=interpret,
    )(x, wt)
    return weights, indices


# speedup vs baseline: 1.3817x; 1.0693x over previous
import functools

import jax
import jax.numpy as jnp
from jax.experimental import pallas as pl

DIM = 2048
N_EXPERTS = 8
BLOCK_N = 2048


def _gate_block(x_ref, wt_ref, w_ref, i_ref):
    bn = x_ref.shape[0]
    scores = jnp.dot(x_ref[...], wt_ref[...], preferred_element_type=jnp.float32)
    m = jnp.max(scores, axis=1, keepdims=True)
    e = jnp.exp(scores - m)
    p = e / jnp.sum(e, axis=1, keepdims=True)
    lane = jax.lax.broadcasted_iota(jnp.int32, (bn, N_EXPERTS), 1)
    w1 = jnp.max(p, axis=1, keepdims=True)
    i1 = jnp.min(jnp.where(p == w1, lane, N_EXPERTS), axis=1, keepdims=True)
    q = jnp.where(lane == i1, -1.0, p)
    w2 = jnp.max(q, axis=1, keepdims=True)
    i2 = jnp.min(jnp.where(q == w2, lane, N_EXPERTS), axis=1, keepdims=True)
    w_ref[...] = jnp.concatenate([w1, w2], axis=1)
    i_ref[...] = jnp.concatenate([i1, i2], axis=1)


@functools.partial(jax.jit, static_argnames=("interpret",))
def kernel(x, W, interpret=False):
    n_tokens = x.shape[0]
    wt = W.T
    grid = (n_tokens // BLOCK_N,)
    weights, indices = pl.pallas_call(
        _gate_block,
        grid=grid,
        in_specs=[
            pl.BlockSpec((BLOCK_N, DIM), lambda i: (i, 0)),
            pl.BlockSpec((DIM, N_EXPERTS), lambda i: (0, 0)),
        ],
        out_specs=[
            pl.BlockSpec((BLOCK_N, 2), lambda i: (i, 0)),
            pl.BlockSpec((BLOCK_N, 2), lambda i: (i, 0)),
        ],
        out_shape=[
            jax.ShapeDtypeStruct((n_tokens, 2), jnp.float32),
            jax.ShapeDtypeStruct((n_tokens, 2), jnp.int32),
        ],
        interpret=interpret,
    )(x, wt)
    return weights, indices
